# Initial kernel scaffold; baseline (speedup 1.0000x reference)
import jax
import jax.numpy as jnp
from jax.experimental import pallas as pl


def kernel(in_feat, in_pos, out_pos, W1, b1, W2, b2, W3, b3, Wd1, bd1, Wd2, bd2, Wd3, bd3):
    raise NotImplementedError("write your pallas kernel here")


# speedup vs baseline: 5.1223x; 5.1223x over previous
import functools

import jax
import jax.numpy as jnp
from jax import lax
from jax.experimental import pallas as pl
from jax.experimental.pallas import tpu as pltpu
from jax.experimental.pallas import tpu_sc as plsc

_NC, _NS = 2, 16
_NW = _NC * _NS
_NPAD = 51200
_NPT = _NPAD // _NS
_NJ_S = _NPT // 128
_BPAD = 53248
_BPW = _BPAD // _NW
_NJ_G = _BPW // 128


def _sc_scatter(featT, idx3, zeros_h, *, P8, cw, n_chunks):
    n_pass = n_chunks // 2
    q = P8 // _NS
    full, rem = divmod(q, _NPT)
    mesh = plsc.VectorSubcoreMesh(core_axis_name="c", subcore_axis_name="s")

    @functools.partial(
        pl.kernel, mesh=mesh,
        compiler_params=pltpu.CompilerParams(use_tc_tiling_on_sc=False),
        out_type=jax.ShapeDtypeStruct((n_chunks, P8, cw), jnp.float32),
        scratch_types=[
            pltpu.VMEM((_NPT, cw), jnp.float32),
            pltpu.VMEM((_NJ_S, 128), jnp.int32),
            pltpu.VMEM_SHARED((P8, cw), jnp.float32),
        ],
    )
    def k(featT_h, idx3_h, zeros_hh, out_h, feat_v, idx2, grid_sh):
        c = lax.axis_index("c")
        s = lax.axis_index("s")
        r0 = s * q
        pltpu.sync_copy(idx3_h.at[s], idx2)
        for p in range(n_pass):
            ch = c * n_pass + p
            pltpu.sync_copy(zeros_hh, feat_v)
            for j in range(full):
                pltpu.sync_copy(feat_v, grid_sh.at[pl.ds(r0 + j * _NPT, _NPT)])
            if rem:
                pltpu.sync_copy(feat_v.at[pl.ds(0, rem)],
                                grid_sh.at[pl.ds(r0 + full * _NPT, rem)])
            pltpu.sync_copy(featT_h.at[ch, pl.ds(s * _NPT, _NPT)], feat_v)
            plsc.subcore_barrier()

            def sbody(j, _):
                pltpu.sync_copy(feat_v.at[pl.ds(j * 128, 128)],
                                grid_sh.at[idx2.at[j]], add=True)
                return _
            lax.fori_loop(0, _NJ_S, sbody, 0)
            plsc.subcore_barrier()
            pltpu.sync_copy(grid_sh.at[pl.ds(r0, q)], out_h.at[ch, pl.ds(r0, q)])
            plsc.subcore_barrier()

    return k(featT, idx3, zeros_h)


def _sc_gather(table, idx3, *, D):
    mesh = plsc.VectorSubcoreMesh(core_axis_name="c", subcore_axis_name="s")

    @functools.partial(
        pl.kernel, mesh=mesh,
        compiler_params=pltpu.CompilerParams(use_tc_tiling_on_sc=False),
        out_type=jax.ShapeDtypeStruct((_BPAD, D), jnp.float32),
        scratch_types=[
            pltpu.VMEM((_NJ_G, 128), jnp.int32),
            pltpu.VMEM((_BPW, D), jnp.float32),
            pltpu.SemaphoreType.DMA,
        ],
    )
    def k(table_h, idx3_h, out_h, idx2, rows_v, sem):
        c = lax.axis_index("c")
        s = lax.axis_index("s")
        wid = s * _NC + c
        pltpu.sync_copy(idx3_h.at[wid], idx2)
        handles = [
            pltpu.async_copy(table_h.at[idx2.at[j]],
                             rows_v.at[pl.ds(j * 128, 128)], sem)
            for j in range(_NJ_G)
        ]
        for h in handles:
            h.wait()
        pltpu.sync_copy(rows_v, out_h.at[pl.ds(wid * _BPW, _BPW)])

    return k(table, idx3)


def _conv_body(prev_ref, cur_ref, nxt_ref, w_ref, b_ref, o_ref, *, T, Gp, relu):
    window = jnp.concatenate([prev_ref[...], cur_ref[...], nxt_ref[...]], axis=0)
    Cout = o_ref.shape[1]
    acc = jnp.zeros((T, Cout), jnp.float32)
    k = 0
    for dx in (-1, 0, 1):
        for dy in (-1, 0, 1):
            for dz in (-1, 0, 1):
                off = (dx * Gp + dy) * Gp + dz
                sl = jax.lax.slice_in_dim(window, T + off, 2 * T + off, axis=0)
                acc = acc + jax.lax.dot_general(
                    sl, w_ref[k], (((1,), (0,)), ((), ())),
                    preferred_element_type=jnp.float32)
                k += 1
    acc = acc + b_ref[...]
    if relu:
        acc = jnp.maximum(acc, 0.0)
    o_ref[...] = acc


def _grid_conv(ext, W, b, *, T, Gp, NB, relu):
    E, Cin = ext.shape
    Cout = W.shape[2]
    body = functools.partial(_conv_body, T=T, Gp=Gp, relu=relu)
    return pl.pallas_call(
        body,
        grid=(NB,),
        in_specs=[
            pl.BlockSpec((T, Cin), lambda i: (i, 0)),
            pl.BlockSpec((T, Cin), lambda i: (i + 1, 0)),
            pl.BlockSpec((T, Cin), lambda i: (i + 2, 0)),
            pl.BlockSpec((27, Cin, Cout), lambda i: (0, 0, 0)),
            pl.BlockSpec((1, Cout), lambda i: (0, 0)),
        ],
        out_specs=pl.BlockSpec((T, Cout), lambda i: (i, 0)),
        out_shape=jax.ShapeDtypeStruct((NB * T, Cout), jnp.float32),
    )(ext, ext, ext, W, b.reshape(1, -1))


def _geom(G):
    Gp = G + 3
    P = Gp ** 3
    S = Gp * Gp + Gp + 1
    T = ((S + 7) // 8) * 8
    NB = -(-P // T)
    return Gp, P, S, T, NB


def _indices(pos, v, G, clip):
    vox = jnp.floor(pos / v).astype(jnp.int32)
    if clip:
        vox = jnp.clip(vox, 0, G - 1)
    Gp = G + 3
    return ((vox[:, 0] + 1) * Gp + (vox[:, 1] + 1)) * Gp + (vox[:, 2] + 1)


def _layer(x, idx3_in, idx3_out, W, b, *, G, cw, relu):
    Gp, P, S, T, NB = _geom(G)
    E = (NB + 2) * T
    C = x.shape[1]
    n_chunks = C // cw
    P8 = ((P + 127) // 128) * 128
    featT = x.reshape(_NPAD, n_chunks, cw).transpose(1, 0, 2)
    zeros_h = jnp.zeros((_NPT, cw), jnp.float32)
    gridc = _sc_scatter(featT, idx3_in, zeros_h, P8=P8, cw=cw, n_chunks=n_chunks)
    grid = gridc.transpose(1, 0, 2).reshape(P8, C)[:P]
    ext = jnp.pad(grid, ((T, E - T - P), (0, 0)))
    out = _grid_conv(ext, W, b, T=T, Gp=Gp, NB=NB, relu=relu)
    return _sc_gather(out, idx3_out, D=W.shape[2])


def kernel(in_feat, in_pos, out_pos, W1, b1, W2, b2, W3, b3,
           Wd1, bd1, Wd2, bd2, Wd3, bd3):
    N = in_feat.shape[0]
    vG = [(0.02, 50), (0.04, 25), (0.08, 13)]
    idx3_in = {}
    idx3_out = {}
    for v, G in vG:
        ii = _indices(in_pos, v, G, True)
        idx3_in[G] = jnp.pad(ii, ((0, _NPAD - N),)).reshape(_NS, _NJ_S, 128)
        io = _indices(out_pos, v, G, False)
        idx3_out[G] = jnp.pad(io, ((0, _BPAD - N),)).reshape(_NW, _NJ_G, 128)

    mask = (jnp.arange(_NPAD) < N)[:, None].astype(jnp.float32)

    x = jnp.pad(in_feat, ((0, _NPAD - N), (0, 13)))
    W1p = jnp.pad(W1, ((0, 0), (0, 13), (0, 0)))
    x = _layer(x, idx3_in[50], idx3_out[50], W1p, b1, G=50, cw=8, relu=True)
    x = x[:_NPAD] * mask
    x = _layer(x, idx3_in[25], idx3_out[25], W2, b2, G=25, cw=16, relu=True)
    x = x[:_NPAD] * mask
    x = _layer(x, idx3_in[13], idx3_out[13], W3, b3, G=13, cw=32, relu=True)
    x = x[:_NPAD] * mask
    x = _layer(x, idx3_in[13], idx3_out[13], Wd1, bd1, G=13, cw=32, relu=True)
    x = x[:_NPAD] * mask
    x = _layer(x, idx3_in[25], idx3_out[25], Wd2, bd2, G=25, cw=16, relu=True)
    x = x[:_NPAD] * mask
    Wd3p = jnp.pad(Wd3, ((0, 0), (0, 0), (0, 3)))
    bd3p = jnp.pad(bd3, ((0, 3),))
    x = _layer(x, idx3_in[50], idx3_out[50], Wd3p, bd3p, G=50, cw=8, relu=False)
    return x[:N, :13]


# speedup vs baseline: 8.1071x; 1.5827x over previous
import functools

import jax
import jax.numpy as jnp
from jax import lax
from jax.experimental import pallas as pl
from jax.experimental.pallas import tpu as pltpu
from jax.experimental.pallas import tpu_sc as plsc

_NC, _NS = 2, 16
_NW = _NC * _NS
_NPAD = 51200
_NPT = _NPAD // _NS
_NJ_S = _NPT // 128
_BPAD = 53248
_BPW = _BPAD // _NW
_NJ_G = _BPW // 128


def _sc_scatter(x2d, idx3, zeros_h, *, P8, cw, n_chunks, T, E):
    n_pass = n_chunks // 2
    q = P8 // _NS
    full, rem = divmod(q, _NPT)
    head, tail = T, E - T - P8
    C = n_chunks * cw
    mesh = plsc.VectorSubcoreMesh(core_axis_name="c", subcore_axis_name="s")

    @functools.partial(
        pl.kernel, mesh=mesh,
        compiler_params=pltpu.CompilerParams(use_tc_tiling_on_sc=False),
        out_type=jax.ShapeDtypeStruct((E, C), jnp.float32),
        scratch_types=[
            pltpu.VMEM((_NPT, cw), jnp.float32),
            pltpu.VMEM((_NJ_S, 128), jnp.int32),
            pltpu.VMEM_SHARED((P8, cw), jnp.float32),
        ],
    )
    def k(x_h, idx3_h, zeros_hh, out_h, feat_v, idx2, grid_sh):
        c = lax.axis_index("c")
        s = lax.axis_index("s")
        r0 = s * q
        pltpu.sync_copy(idx3_h.at[s], idx2)
        for p in range(n_pass):
            ch = c * n_pass + p
            c0 = ch * cw
            pltpu.sync_copy(zeros_hh, feat_v)
            for j in range(full):
                pltpu.sync_copy(feat_v, grid_sh.at[pl.ds(r0 + j * _NPT, _NPT)])
            if rem:
                pltpu.sync_copy(feat_v.at[pl.ds(0, rem)],
                                grid_sh.at[pl.ds(r0 + full * _NPT, rem)])
            @pl.when(s == 0)
            def _():
                pltpu.sync_copy(feat_v.at[pl.ds(0, head)],
                                out_h.at[pl.ds(0, head), pl.ds(c0, cw)])

            @pl.when(s == 1)
            def _():
                pltpu.sync_copy(feat_v.at[pl.ds(0, tail)],
                                out_h.at[pl.ds(T + P8, tail), pl.ds(c0, cw)])
            pltpu.sync_copy(x_h.at[pl.ds(s * _NPT, _NPT), pl.ds(c0, cw)], feat_v)
            plsc.subcore_barrier()

            def sbody(j, _):
                pltpu.sync_copy(feat_v.at[pl.ds(j * 128, 128)],
                                grid_sh.at[idx2.at[j]], add=True)
                return _
            lax.fori_loop(0, _NJ_S, sbody, 0)
            plsc.subcore_barrier()
            pltpu.sync_copy(grid_sh.at[pl.ds(r0, q)],
                            out_h.at[pl.ds(T + r0, q), pl.ds(c0, cw)])
            plsc.subcore_barrier()

    return k(x2d, idx3, zeros_h)


def _sc_gather(table, idx3, *, D):
    mesh = plsc.VectorSubcoreMesh(core_axis_name="c", subcore_axis_name="s")

    @functools.partial(
        pl.kernel, mesh=mesh,
        compiler_params=pltpu.CompilerParams(use_tc_tiling_on_sc=False),
        out_type=jax.ShapeDtypeStruct((_BPAD, D), jnp.float32),
        scratch_types=[
            pltpu.VMEM((_NJ_G, 128), jnp.int32),
            pltpu.VMEM((_BPW, D), jnp.float32),
            pltpu.SemaphoreType.DMA,
        ],
    )
    def k(table_h, idx3_h, out_h, idx2, rows_v, sem):
        c = lax.axis_index("c")
        s = lax.axis_index("s")
        wid = s * _NC + c
        pltpu.sync_copy(idx3_h.at[wid], idx2)
        handles = [
            pltpu.async_copy(table_h.at[idx2.at[j]],
                             rows_v.at[pl.ds(j * 128, 128)], sem)
            for j in range(_NJ_G)
        ]
        for h in handles:
            h.wait()
        pltpu.sync_copy(rows_v, out_h.at[pl.ds(wid * _BPW, _BPW)])

    return k(table, idx3)


def _conv_body(prev_ref, cur_ref, nxt_ref, w_ref, b_ref, o_ref, *, T, Gp, relu):
    window = jnp.concatenate([prev_ref[...], cur_ref[...], nxt_ref[...]], axis=0)
    Cout = o_ref.shape[1]
    acc = jnp.zeros((T, Cout), jnp.float32)
    k = 0
    for dx in (-1, 0, 1):
        for dy in (-1, 0, 1):
            for dz in (-1, 0, 1):
                off = (dx * Gp + dy) * Gp + dz
                sl = jax.lax.slice_in_dim(window, T + off, 2 * T + off, axis=0)
                acc = acc + jax.lax.dot_general(
                    sl, w_ref[k], (((1,), (0,)), ((), ())),
                    preferred_element_type=jnp.float32)
                k += 1
    acc = acc + b_ref[...]
    if relu:
        acc = jnp.maximum(acc, 0.0)
    o_ref[...] = acc


def _grid_conv(ext, W, b, *, T, Gp, NB, relu):
    E, Cin = ext.shape
    Cout = W.shape[2]
    body = functools.partial(_conv_body, T=T, Gp=Gp, relu=relu)
    return pl.pallas_call(
        body,
        grid=(NB,),
        in_specs=[
            pl.BlockSpec((T, Cin), lambda i: (i, 0)),
            pl.BlockSpec((T, Cin), lambda i: (i + 1, 0)),
            pl.BlockSpec((T, Cin), lambda i: (i + 2, 0)),
            pl.BlockSpec((27, Cin, Cout), lambda i: (0, 0, 0)),
            pl.BlockSpec((1, Cout), lambda i: (0, 0)),
        ],
        out_specs=pl.BlockSpec((T, Cout), lambda i: (i, 0)),
        out_shape=jax.ShapeDtypeStruct((NB * T, Cout), jnp.float32),
    )(ext, ext, ext, W, b.reshape(1, -1))


def _geom(G):
    Gp = G + 3
    P = Gp ** 3
    S = Gp * Gp + Gp + 1
    T = ((S + 7) // 8) * 8
    NB = -(-P // T)
    return Gp, P, S, T, NB


def _indices(pos, v, G, clip):
    vox = jnp.floor(pos / v).astype(jnp.int32)
    if clip:
        vox = jnp.clip(vox, 0, G - 1)
    Gp = G + 3
    return ((vox[:, 0] + 1) * Gp + (vox[:, 1] + 1)) * Gp + (vox[:, 2] + 1)


def _layer(x, idx3_in, idx3_out, W, b, *, G, cw, relu):
    Gp, P, S, T, NB = _geom(G)
    E = (NB + 2) * T
    C = x.shape[1]
    n_chunks = C // cw
    P8 = ((P + 127) // 128) * 128
    zeros_h = jnp.zeros((_NPT, cw), jnp.float32)
    ext = _sc_scatter(x, idx3_in, zeros_h, P8=P8, cw=cw, n_chunks=n_chunks,
                      T=T, E=E)
    out = _grid_conv(ext, W, b, T=T, Gp=Gp, NB=NB, relu=relu)
    return _sc_gather(out, idx3_out, D=W.shape[2])


def kernel(in_feat, in_pos, out_pos, W1, b1, W2, b2, W3, b3,
           Wd1, bd1, Wd2, bd2, Wd3, bd3):
    N = in_feat.shape[0]
    vG = [(0.02, 50), (0.04, 25), (0.08, 13)]
    idx3_in = {}
    idx3_out = {}
    for v, G in vG:
        ii = _indices(in_pos, v, G, True)
        idx3_in[G] = jnp.pad(ii, ((0, _NPAD - N),)).reshape(_NS, _NJ_S, 128)
        io = _indices(out_pos, v, G, False)
        idx3_out[G] = jnp.pad(io, ((0, _BPAD - N),)).reshape(_NW, _NJ_G, 128)

    mask = (jnp.arange(_NPAD) < N)[:, None].astype(jnp.float32)

    x = jnp.pad(in_feat, ((0, _NPAD - N), (0, 13)))
    W1p = jnp.pad(W1, ((0, 0), (0, 13), (0, 0)))
    x = _layer(x, idx3_in[50], idx3_out[50], W1p, b1, G=50, cw=8, relu=True)
    x = x[:_NPAD] * mask
    x = _layer(x, idx3_in[25], idx3_out[25], W2, b2, G=25, cw=16, relu=True)
    x = x[:_NPAD] * mask
    x = _layer(x, idx3_in[13], idx3_out[13], W3, b3, G=13, cw=32, relu=True)
    x = x[:_NPAD] * mask
    x = _layer(x, idx3_in[13], idx3_out[13], Wd1, bd1, G=13, cw=32, relu=True)
    x = x[:_NPAD] * mask
    x = _layer(x, idx3_in[25], idx3_out[25], Wd2, bd2, G=25, cw=16, relu=True)
    x = x[:_NPAD] * mask
    Wd3p = jnp.pad(Wd3, ((0, 0), (0, 0), (0, 3)))
    bd3p = jnp.pad(bd3, ((0, 3),))
    x = _layer(x, idx3_in[50], idx3_out[50], Wd3p, bd3p, G=50, cw=8, relu=False)
    return x[:N, :13]


# speedup vs baseline: 9.4353x; 1.1638x over previous
import functools

import jax
import jax.numpy as jnp
from jax import lax
from jax.experimental import pallas as pl
from jax.experimental.pallas import tpu as pltpu
from jax.experimental.pallas import tpu_sc as plsc

_NC, _NS = 2, 16
_NW = _NC * _NS
_NPAD = 51200
_NPT = _NPAD // _NS
_NJ_S = _NPT // 128
_BPAD = 53248
_BPW = _BPAD // _NW
_NJ_G = _BPW // 128


def _sc_scatter(x2d, idx3, zeros_h, *, P8, cw, n_chunks, T, E):
    n_pass = n_chunks // 2
    q = P8 // _NS
    full, rem = divmod(q, _NPT)
    head, tail = T, E - T - P8
    C = n_chunks * cw
    mesh = plsc.VectorSubcoreMesh(core_axis_name="c", subcore_axis_name="s")

    @functools.partial(
        pl.kernel, mesh=mesh,
        compiler_params=pltpu.CompilerParams(use_tc_tiling_on_sc=False),
        out_type=jax.ShapeDtypeStruct((E, C), jnp.float32),
        scratch_types=[
            pltpu.VMEM((_NPT, cw), jnp.float32),
            pltpu.VMEM((_NJ_S, 128), jnp.int32),
            pltpu.VMEM_SHARED((P8, cw), jnp.float32),
        ],
    )
    def k(x_h, idx3_h, zeros_hh, out_h, feat_v, idx2, grid_sh):
        c = lax.axis_index("c")
        s = lax.axis_index("s")
        r0 = s * q
        pltpu.sync_copy(idx3_h.at[s], idx2)
        for p in range(n_pass):
            ch = c * n_pass + p
            c0 = ch * cw
            pltpu.sync_copy(zeros_hh, feat_v)
            for j in range(full):
                pltpu.sync_copy(feat_v, grid_sh.at[pl.ds(r0 + j * _NPT, _NPT)])
            if rem:
                pltpu.sync_copy(feat_v.at[pl.ds(0, rem)],
                                grid_sh.at[pl.ds(r0 + full * _NPT, rem)])
            @pl.when(s == 0)
            def _():
                pltpu.sync_copy(feat_v.at[pl.ds(0, head)],
                                out_h.at[pl.ds(0, head), pl.ds(c0, cw)])

            @pl.when(s == 1)
            def _():
                pltpu.sync_copy(feat_v.at[pl.ds(0, tail)],
                                out_h.at[pl.ds(T + P8, tail), pl.ds(c0, cw)])
            pltpu.sync_copy(x_h.at[pl.ds(s * _NPT, _NPT), pl.ds(c0, cw)], feat_v)
            plsc.subcore_barrier()

            def sbody(j, _):
                pltpu.sync_copy(feat_v.at[pl.ds(j * 128, 128)],
                                grid_sh.at[idx2.at[j]], add=True)
                return _
            lax.fori_loop(0, _NJ_S, sbody, 0)
            plsc.subcore_barrier()
            pltpu.sync_copy(grid_sh.at[pl.ds(r0, q)],
                            out_h.at[pl.ds(T + r0, q), pl.ds(c0, cw)])
            plsc.subcore_barrier()

    return k(x2d, idx3, zeros_h)


def _sc_gather_scatter(table, idx3g, idx3s, zeros_h, *, D, P8, cw, n_chunks,
                       T, E):
    q = P8 // _NS
    head, tail = T, E - T - P8
    C = n_chunks * cw
    assert C == D and table.shape[0] == n_chunks
    mesh = plsc.VectorSubcoreMesh(core_axis_name="c", subcore_axis_name="s")

    @functools.partial(
        pl.kernel, mesh=mesh,
        compiler_params=pltpu.CompilerParams(use_tc_tiling_on_sc=False),
        out_type=jax.ShapeDtypeStruct((2, E, C), jnp.float32),
        scratch_types=[
            pltpu.VMEM((_BPW, cw), jnp.float32),
            pltpu.VMEM((_NJ_G, 128), jnp.int32),
            pltpu.VMEM((_NJ_G, 128), jnp.int32),
            pltpu.VMEM_SHARED((P8, cw), jnp.float32),
            pltpu.SemaphoreType.DMA,
        ],
    )
    def k(table_h, idx3g_h, idx3s_h, zeros_hh, out_h, chunk_v, idx2g, idx2s,
          grid_sh, sem):
        c = lax.axis_index("c")
        s = lax.axis_index("s")
        wid = s * _NC + c
        r0 = s * q
        pltpu.sync_copy(idx3g_h.at[wid], idx2g)
        pltpu.sync_copy(idx3s_h.at[wid], idx2s)
        for ch in range(n_chunks):
            c0 = ch * cw
            handles = [
                pltpu.async_copy(table_h.at[ch].at[idx2g.at[j]],
                                 chunk_v.at[pl.ds(j * 128, 128)], sem)
                for j in range(_NJ_G)
            ]
            pltpu.sync_copy(zeros_hh.at[pl.ds(0, q)],
                            grid_sh.at[pl.ds(r0, q)])
            @pl.when(s == 0)
            def _():
                pltpu.sync_copy(zeros_hh.at[pl.ds(0, head)],
                                out_h.at[c, pl.ds(0, head), pl.ds(c0, cw)])

            @pl.when(s == 1)
            def _():
                pltpu.sync_copy(zeros_hh.at[pl.ds(0, tail)],
                                out_h.at[c, pl.ds(T + P8, tail), pl.ds(c0, cw)])
            for h in handles:
                h.wait()
            plsc.subcore_barrier()

            def sbody(j, _):
                pltpu.sync_copy(chunk_v.at[pl.ds(j * 128, 128)],
                                grid_sh.at[idx2s.at[j]], add=True)
                return _
            lax.fori_loop(0, _NJ_G, sbody, 0)
            plsc.subcore_barrier()
            pltpu.sync_copy(grid_sh.at[pl.ds(r0, q)],
                            out_h.at[c, pl.ds(T + r0, q), pl.ds(c0, cw)])
            plsc.subcore_barrier()

    return k(table, idx3g, idx3s, zeros_h)


def _sc_gather(table, idx3, *, D):
    mesh = plsc.VectorSubcoreMesh(core_axis_name="c", subcore_axis_name="s")

    @functools.partial(
        pl.kernel, mesh=mesh,
        compiler_params=pltpu.CompilerParams(use_tc_tiling_on_sc=False),
        out_type=jax.ShapeDtypeStruct((_BPAD, D), jnp.float32),
        scratch_types=[
            pltpu.VMEM((_NJ_G, 128), jnp.int32),
            pltpu.VMEM((_BPW, D), jnp.float32),
            pltpu.SemaphoreType.DMA,
        ],
    )
    def k(table_h, idx3_h, out_h, idx2, rows_v, sem):
        c = lax.axis_index("c")
        s = lax.axis_index("s")
        wid = s * _NC + c
        pltpu.sync_copy(idx3_h.at[wid], idx2)
        handles = [
            pltpu.async_copy(table_h.at[idx2.at[j]],
                             rows_v.at[pl.ds(j * 128, 128)], sem)
            for j in range(_NJ_G)
        ]
        for h in handles:
            h.wait()
        pltpu.sync_copy(rows_v, out_h.at[pl.ds(wid * _BPW, _BPW)])

    return k(table, idx3)


def _acc_out(acc, o_ref, relu, b_ref, noc):
    acc = acc + b_ref[...]
    if relu:
        acc = jnp.maximum(acc, 0.0)
    if noc == 1:
        o_ref[...] = acc
    else:
        ocw = acc.shape[1] // noc
        for j in range(noc):
            o_ref[j] = jax.lax.slice_in_dim(acc, j * ocw, (j + 1) * ocw, axis=1)


def _conv_body(prev_ref, cur_ref, nxt_ref, w_ref, b_ref, o_ref, *, T, Gp, relu,
               noc):
    window = jnp.concatenate([prev_ref[...], cur_ref[...], nxt_ref[...]], axis=0)
    Cout = w_ref.shape[2]
    acc = jnp.zeros((T, Cout), jnp.float32)
    k = 0
    for dx in (-1, 0, 1):
        for dy in (-1, 0, 1):
            for dz in (-1, 0, 1):
                off = (dx * Gp + dy) * Gp + dz
                sl = jax.lax.slice_in_dim(window, T + off, 2 * T + off, axis=0)
                acc = acc + jax.lax.dot_general(
                    sl, w_ref[k], (((1,), (0,)), ((), ())),
                    preferred_element_type=jnp.float32)
                k += 1
    _acc_out(acc, o_ref, relu, b_ref, noc)


def _ospec(noc, Cout, T, NB):
    if noc == 1:
        return (pl.BlockSpec((T, Cout), lambda i: (i, 0)),
                jax.ShapeDtypeStruct((NB * T, Cout), jnp.float32))
    return (pl.BlockSpec((noc, T, Cout // noc), lambda i: (0, i, 0)),
            jax.ShapeDtypeStruct((noc, NB * T, Cout // noc), jnp.float32))


def _grid_conv(ext, W, b, *, T, Gp, NB, relu, noc=1):
    E, Cin = ext.shape
    Cout = W.shape[2]
    body = functools.partial(_conv_body, T=T, Gp=Gp, relu=relu, noc=noc)
    out_spec, out_shape = _ospec(noc, Cout, T, NB)
    return pl.pallas_call(
        body,
        grid=(NB,),
        in_specs=[
            pl.BlockSpec((T, Cin), lambda i: (i, 0)),
            pl.BlockSpec((T, Cin), lambda i: (i + 1, 0)),
            pl.BlockSpec((T, Cin), lambda i: (i + 2, 0)),
            pl.BlockSpec((27, Cin, Cout), lambda i: (0, 0, 0)),
            pl.BlockSpec((1, Cout), lambda i: (0, 0)),
        ],
        out_specs=out_spec,
        out_shape=out_shape,
    )(ext, ext, ext, W, b.reshape(1, -1))


def _conv2_body(pa, ca, na, pb, cb, nb, w_ref, b_ref, o_ref, *, T, Gp, relu,
                noc):
    window = (jnp.concatenate([pa[0], ca[0], na[0]], axis=0)
              + jnp.concatenate([pb[0], cb[0], nb[0]], axis=0))
    Cout = w_ref.shape[2]
    acc = jnp.zeros((T, Cout), jnp.float32)
    k = 0
    for dx in (-1, 0, 1):
        for dy in (-1, 0, 1):
            for dz in (-1, 0, 1):
                off = (dx * Gp + dy) * Gp + dz
                sl = jax.lax.slice_in_dim(window, T + off, 2 * T + off, axis=0)
                acc = acc + jax.lax.dot_general(
                    sl, w_ref[k], (((1,), (0,)), ((), ())),
                    preferred_element_type=jnp.float32)
                k += 1
    _acc_out(acc, o_ref, relu, b_ref, noc)


def _grid_conv2(ext2, W, b, *, T, Gp, NB, relu, noc=1):
    _, E, Cin = ext2.shape
    Cout = W.shape[2]
    body = functools.partial(_conv2_body, T=T, Gp=Gp, relu=relu, noc=noc)
    espec = lambda core, d: pl.BlockSpec((1, T, Cin),
                                         lambda i, core=core, d=d: (core, i + d, 0))
    out_spec, out_shape = _ospec(noc, Cout, T, NB)
    return pl.pallas_call(
        body,
        grid=(NB,),
        in_specs=[
            espec(0, 0), espec(0, 1), espec(0, 2),
            espec(1, 0), espec(1, 1), espec(1, 2),
            pl.BlockSpec((27, Cin, Cout), lambda i: (0, 0, 0)),
            pl.BlockSpec((1, Cout), lambda i: (0, 0)),
        ],
        out_specs=out_spec,
        out_shape=out_shape,
    )(ext2, ext2, ext2, ext2, ext2, ext2, W, b.reshape(1, -1))


def _geom(G):
    Gp = G + 3
    P = Gp ** 3
    S = Gp * Gp + Gp + 1
    T = ((S + 7) // 8) * 8
    NB = -(-P // T)
    P8 = ((P + 128) // 128) * 128
    return Gp, P, S, T, NB, P8


def _indices(pos, v, G, clip):
    vox = jnp.floor(pos / v).astype(jnp.int32)
    if clip:
        vox = jnp.clip(vox, 0, G - 1)
    Gp = G + 3
    return ((vox[:, 0] + 1) * Gp + (vox[:, 1] + 1)) * Gp + (vox[:, 2] + 1)


def kernel(in_feat, in_pos, out_pos, W1, b1, W2, b2, W3, b3,
           Wd1, bd1, Wd2, bd2, Wd3, bd3):
    N = in_feat.shape[0]
    vG = [(0.02, 50), (0.04, 25), (0.08, 13)]
    geo = {G: _geom(G) for _, G in vG}
    idx3_in = {}
    idx3_inB = {}
    idx3_out = {}
    for v, G in vG:
        Gp, P, S, T, NB, P8 = geo[G]
        ii = _indices(in_pos, v, G, True)
        idx3_in[G] = jnp.pad(ii, ((0, _NPAD - N),), constant_values=P
                             ).reshape(_NS, _NJ_S, 128)
        idx3_inB[G] = jnp.pad(ii, ((0, _BPAD - N),), constant_values=P
                              ).reshape(_NW, _NJ_G, 128)
        io = _indices(out_pos, v, G, False)
        idx3_out[G] = jnp.pad(io, ((0, _BPAD - N),)).reshape(_NW, _NJ_G, 128)

    z8 = jnp.zeros((_NPT, 8), jnp.float32)
    z16 = jnp.zeros((_NPT, 16), jnp.float32)
    z32 = jnp.zeros((_NPT, 32), jnp.float32)

    def geoargs(G, cw):
        Gp, P, S, T, NB, P8 = geo[G]
        return dict(P8=P8, cw=cw, T=T, E=(NB + 2) * T)

    def convargs(G):
        Gp, P, S, T, NB, P8 = geo[G]
        return dict(T=T, Gp=Gp, NB=NB)

    x = jnp.pad(in_feat, ((0, _NPAD - N), (0, 13)))
    W1p = jnp.pad(W1, ((0, 0), (0, 13), (0, 0)))
    ext = _sc_scatter(x, idx3_in[50], z8, n_chunks=2, **geoargs(50, 8))
    t = _grid_conv(ext, W1p, b1, relu=True, noc=2, **convargs(50))
    e2 = _sc_gather_scatter(t, idx3_out[50], idx3_inB[25], z16, D=32,
                            n_chunks=2, **geoargs(25, 16))
    t = _grid_conv2(e2, W2, b2, relu=True, noc=2, **convargs(25))
    e2 = _sc_gather_scatter(t, idx3_out[25], idx3_inB[13], z32, D=64,
                            n_chunks=2, **geoargs(13, 32))
    t = _grid_conv2(e2, W3, b3, relu=True, noc=2, **convargs(13))
    e2 = _sc_gather_scatter(t, idx3_out[13], idx3_inB[13], z32, D=64,
                            n_chunks=2, **geoargs(13, 32))
    t = _grid_conv2(e2, Wd1, bd1, relu=True, noc=2, **convargs(13))
    e2 = _sc_gather_scatter(t, idx3_out[13], idx3_inB[25], z32, D=64,
                            n_chunks=2, **geoargs(25, 32))
    t = _grid_conv2(e2, Wd2, bd2, relu=True, **convargs(25))
    x6 = _sc_gather(t, idx3_out[25], D=64)
    ext = _sc_scatter(x6, idx3_in[50], z8, n_chunks=8, **geoargs(50, 8))
    Wd3p = jnp.pad(Wd3, ((0, 0), (0, 0), (0, 3)))
    bd3p = jnp.pad(bd3, ((0, 3),))
    t = _grid_conv(ext, Wd3p, bd3p, relu=False, **convargs(50))
    out = _sc_gather(t, idx3_out[50], D=16)
    return out[:N, :13]
